# Initial kernel scaffold; baseline (speedup 1.0000x reference)
import jax
import jax.numpy as jnp
from jax.experimental import pallas as pl


def kernel(x, edge_index, batch, edge_attr, Wi, bi, W1, b1, W2, b2, Wo, bo):
    raise NotImplementedError("write your pallas kernel here")


# speedup vs baseline: 83.2382x; 83.2382x over previous
import functools

import jax
import jax.numpy as jnp
from jax import lax
from jax.experimental import pallas as pl
from jax.experimental.pallas import tpu as pltpu
from jax.experimental.pallas import tpu_sc as plsc

N_NODES = 100000
N_EDGES = 6400000
NUM_FEATURES = 128
LAT = 10
NUM_PROTS = 1000

DP = 16
NPAD = 100096
NW = 32
TR = 1568
EROWS_P = TR * NW
C = 8
SLAB = NPAD // 16
ZROWS = 368

_mesh = plsc.VectorSubcoreMesh(core_axis_name="c", subcore_axis_name="s")


@functools.partial(
    pl.kernel,
    out_type=jax.ShapeDtypeStruct((2 * NPAD,), jnp.float32),
    mesh=_mesh,
    compiler_params=pltpu.CompilerParams(use_tc_tiling_on_sc=False),
    scratch_types=[
        pltpu.VMEM((C, 128), jnp.int32),
        pltpu.VMEM((128,), jnp.float32),
        pltpu.VMEM((SLAB,), jnp.float32),
        pltpu.VMEM_SHARED((NPAD,), jnp.float32),
        pltpu.SemaphoreType.DMA,
    ],
)
def _deg_kernel(dst_hbm, out_hbm, didx, ones, zbuf, deg_sh, sem):
    cid = lax.axis_index("c")
    sid = lax.axis_index("s")
    wid = cid * 16 + sid

    @pl.loop(0, 128 // 16)
    def _(i):
        ones[pl.ds(i * 16, 16)] = jnp.ones((16,), jnp.float32)

    @pl.loop(0, SLAB // 16)
    def _(i):
        zbuf[pl.ds(i * 16, 16)] = jnp.zeros((16,), jnp.float32)

    pltpu.sync_copy(zbuf, deg_sh.at[pl.ds(sid * SLAB, SLAB)])
    plsc.subcore_barrier()

    base = wid * TR

    @pl.loop(0, TR // C)
    def _(t):
        r0 = base + t * C
        pltpu.sync_copy(dst_hbm.at[pl.ds(r0, C)], didx)
        for j in range(C):
            pltpu.sync_copy(ones, deg_sh.at[didx.at[j]], add=True)

    plsc.subcore_barrier()
    pltpu.sync_copy(deg_sh.at[pl.ds(sid * SLAB, SLAB)], zbuf)
    pltpu.sync_copy(zbuf, out_hbm.at[pl.ds(cid * NPAD + sid * SLAB, SLAB)])


@functools.partial(
    pl.kernel,
    out_type=jax.ShapeDtypeStruct((2, NPAD, DP), jnp.float32),
    mesh=_mesh,
    compiler_params=pltpu.CompilerParams(use_tc_tiling_on_sc=False),
    scratch_types=[
        pltpu.VMEM((C, 128), jnp.int32),
        pltpu.VMEM((C, 128), jnp.int32),
        pltpu.VMEM((C, 128, DP), jnp.float32),
        pltpu.VMEM((ZROWS, DP), jnp.float32),
        pltpu.VMEM_SHARED((NPAD, DP), jnp.float32),
        pltpu.SemaphoreType.DMA,
        pltpu.SemaphoreType.DMA,
    ],
)
def _agg_kernel(g_hbm, src_hbm, dst_hbm, out_hbm,
                sidx, didx, rows, zbuf, acc_sh, gsem, ssem):
    cid = lax.axis_index("c")
    sid = lax.axis_index("s")
    wid = cid * 16 + sid

    @pl.loop(0, ZROWS)
    def _(i):
        zbuf[i] = jnp.zeros((DP,), jnp.float32)

    for k in range(17):
        pltpu.sync_copy(zbuf, acc_sh.at[pl.ds(sid * SLAB + k * ZROWS, ZROWS)])
    plsc.subcore_barrier()

    base = wid * TR

    @pl.loop(0, TR // C)
    def _(t):
        r0 = base + t * C
        pltpu.sync_copy(src_hbm.at[pl.ds(r0, C)], sidx)
        pltpu.sync_copy(dst_hbm.at[pl.ds(r0, C)], didx)
        gd = [pltpu.async_copy(g_hbm.at[sidx.at[j]], rows.at[j], gsem)
              for j in range(C)]
        for d in gd:
            d.wait()
        sd = [pltpu.async_copy(rows.at[j], acc_sh.at[didx.at[j]], ssem,
                               add=True)
              for j in range(C)]
        for d in sd:
            d.wait()

    plsc.subcore_barrier()
    for k in range(17):
        r0 = sid * SLAB + k * ZROWS
        pltpu.sync_copy(acc_sh.at[pl.ds(r0, ZROWS)], zbuf)
        pltpu.sync_copy(zbuf, out_hbm.at[cid, pl.ds(r0, ZROWS)])


def _k2_body(x_ref, dga_ref, dgb_ref, wi_ref, bi_ref, w1_ref,
             g1_ref, dinv_ref):
    h0 = jnp.maximum(
        jnp.dot(x_ref[...], wi_ref[...], preferred_element_type=jnp.float32)
        + bi_ref[0, :], 0.0)
    xw1 = jnp.dot(h0, w1_ref[...], preferred_element_type=jnp.float32)
    dinv = lax.rsqrt(dga_ref[...] + dgb_ref[...] + 2.0)
    g1_ref[...] = dinv * xw1
    dinv_ref[...] = dinv


def _k4_body(acca_ref, accb_ref, g_ref, dinv_ref, w_ref, b_ref, g2_ref):
    acc = acca_ref[...] + accb_ref[...] + 2.0 * g_ref[...]
    h = jnp.maximum(dinv_ref[...] * acc + b_ref[0, :], 0.0)
    xw = jnp.dot(h, w_ref[...], preferred_element_type=jnp.float32)
    g2_ref[...] = dinv_ref[...] * xw


def _k6_body(acca_ref, accb_ref, g_ref, dinv_ref, wot_ref, b_ref, bo_ref,
             out_ref):
    acc = acca_ref[...] + accb_ref[...] + 2.0 * g_ref[...]
    h = jnp.maximum(dinv_ref[...] * acc + b_ref[0, :], 0.0)
    out_ref[...] = (jnp.sum(h * wot_ref[...], axis=1, keepdims=True)
                    + bo_ref[0, 0])


_RB = 2000
_GRID = N_NODES // _RB

_row = lambda i: (i, 0)
_rep = lambda i: (0, 0)
_b_x = pl.BlockSpec((_RB, NUM_FEATURES), _row)
_b_lat = pl.BlockSpec((_RB, DP), _row)
_b_col = pl.BlockSpec((_RB, 1), _row)
_b_wi = pl.BlockSpec((NUM_FEATURES, DP), _rep)
_b_w = pl.BlockSpec((DP, DP), _rep)
_b_brow = pl.BlockSpec((1, DP), _rep)
_b_scal = pl.BlockSpec((1, 1), _rep)


def kernel(x, edge_index, batch, edge_attr, Wi, bi, W1, b1, W2, b2, Wo, bo):
    del batch, edge_attr
    f32 = jnp.float32
    src = edge_index[0].astype(jnp.int32)
    dst = edge_index[1].astype(jnp.int32)

    pad_n = EROWS_P * 128 - N_EDGES
    spread = (jnp.arange(pad_n, dtype=jnp.int32) % 64) + N_NODES
    src_p = jnp.concatenate([src, spread]).reshape(EROWS_P, 128)
    dst_p = jnp.concatenate([dst, spread]).reshape(EROWS_P, 128)

    wi_p = jnp.zeros((NUM_FEATURES, DP), f32).at[:, :LAT].set(Wi)
    bi_p = jnp.zeros((1, DP), f32).at[0, :LAT].set(bi)
    w1_p = jnp.zeros((DP, DP), f32).at[:LAT, :LAT].set(W1)
    b1_p = jnp.zeros((1, DP), f32).at[0, :LAT].set(b1)
    w2_p = jnp.zeros((DP, DP), f32).at[:LAT, :LAT].set(W2)
    b2_p = jnp.zeros((1, DP), f32).at[0, :LAT].set(b2)
    wot = jnp.zeros((1, DP), f32).at[0, :LAT].set(Wo[:, 0])
    bo_p = bo.reshape(1, 1)

    deg2 = _deg_kernel(dst_p)
    dga = deg2[:N_NODES].reshape(N_NODES, 1)
    dgb = deg2[NPAD:NPAD + N_NODES].reshape(N_NODES, 1)

    g1, dinv = pl.pallas_call(
        _k2_body,
        grid=(_GRID,),
        in_specs=[_b_x, _b_col, _b_col, _b_wi, _b_brow, _b_w],
        out_specs=[_b_lat, _b_col],
        out_shape=[jax.ShapeDtypeStruct((N_NODES, DP), f32),
                   jax.ShapeDtypeStruct((N_NODES, 1), f32)],
    )(x, dga, dgb, wi_p, bi_p, w1_p)

    g1_t = jnp.zeros((NPAD, DP), f32).at[:N_NODES].set(g1)
    acc1 = _agg_kernel(g1_t, src_p, dst_p)

    g2 = pl.pallas_call(
        _k4_body,
        grid=(_GRID,),
        in_specs=[_b_lat, _b_lat, _b_lat, _b_col, _b_w, _b_brow],
        out_specs=_b_lat,
        out_shape=jax.ShapeDtypeStruct((N_NODES, DP), f32),
    )(acc1[0, :N_NODES], acc1[1, :N_NODES], g1, dinv, w2_p, b1_p)

    g2_t = jnp.zeros((NPAD, DP), f32).at[:N_NODES].set(g2)
    acc2 = _agg_kernel(g2_t, src_p, dst_p)

    out = pl.pallas_call(
        _k6_body,
        grid=(_GRID,),
        in_specs=[_b_lat, _b_lat, _b_lat, _b_col, _b_brow, _b_brow, _b_scal],
        out_specs=_b_col,
        out_shape=jax.ShapeDtypeStruct((N_NODES, 1), f32),
    )(acc2[0, :N_NODES], acc2[1, :N_NODES], g2, dinv, wot, b2_p, bo_p)

    return out.reshape(-1, NUM_PROTS, 1)
